# baseline (speedup 1.0000x reference)
import jax
import jax.numpy as jnp
from jax.experimental import pallas as pl
from jax.experimental.pallas import tpu as pltpu
from jax.experimental.pallas import tpu_sc as plsc

N = 4096
H = 4
D = 128
B = 1024
K = 8
BBLK = 256


def _sc_gather_rows(table, idx, value_dim, window):
    num = idx.shape[0]
    idx2 = idx.reshape(1, num)
    mesh = plsc.VectorSubcoreMesh(core_axis_name="c", subcore_axis_name="s")

    @pl.kernel(
        out_type=jax.ShapeDtypeStruct((num, value_dim), table.dtype),
        mesh=mesh,
    )
    def gather_kernel(x_hbm, i_hbm, o_hbm):
        def body(i_vmem, o_vmem):
            pltpu.sync_copy(x_hbm.at[i_vmem.at[0]], o_vmem)

        pltpu.emit_pipeline(
            body,
            grid=(num // window,),
            in_specs=[pl.BlockSpec((1, window), lambda i: (0, i))],
            out_specs=[pl.BlockSpec((window, value_dim), lambda i: (i, 0))],
            core_axis_name=("c", "s"),
            dimension_semantics=(pltpu.PARALLEL,),
        )(i_hbm, o_hbm)

    return gather_kernel(table, idx2)


def _chunked_idx(idx, chunks):
    return (idx[:, None] * chunks
            + jnp.arange(chunks, dtype=jnp.int32)).reshape(-1)


def _topk_body(pos_ref, q_ref, g_ref, al_ref, lb_out, ct_out):
    pos_h = pos_ref[0]
    q = q_ref[0]
    g = g_ref[0]
    acol = al_ref[...]

    dn = (((1,), (1,)), ((), ()))
    cross = jax.lax.dot_general(q, pos_h, dn, preferred_element_type=jnp.float32)
    ones = jnp.ones((1, D), jnp.float32)
    pn = jax.lax.dot_general(ones, pos_h * pos_h, dn,
                             preferred_element_type=jnp.float32)
    qn = jnp.sum(q * q, axis=1, keepdims=True)
    qg = jnp.sum(q * g, axis=1, keepdims=True)

    vals = pn - 2.0 * cross
    inf = jnp.float32(jnp.inf)
    w = 256
    f1 = vals[:, :w]
    f2 = jnp.full((BBLK, w), inf, jnp.float32)
    for i in range(1, N // w):
        sl = vals[:, i * w:(i + 1) * w]
        f2 = jnp.minimum(f2, jnp.maximum(f1, sl))
        f1 = jnp.minimum(f1, sl)
    cand = jnp.concatenate([f1, f2], axis=1)
    mprev = jnp.min(cand, axis=1, keepdims=True)
    lo = None
    for r in range(1, K + 1):
        mprev = jnp.min(jnp.where(cand > mprev, cand, inf), axis=1,
                        keepdims=True)
        if r == 1:
            lo = mprev
    msel = (vals >= lo) & (vals <= mprev)
    e = jnp.where(msel, jnp.exp(-jnp.sqrt(jnp.maximum(vals + qn, 0.0))), 0.0)
    s = jnp.sum(e, axis=1, keepdims=True)
    t = jnp.sum(e * acol, axis=1, keepdims=True)
    p = jax.lax.dot_general(e, pos_h, (((1,), (0,)), ((), ())),
                            preferred_element_type=jnp.float32)
    pg = jnp.sum(p * g, axis=1, keepdims=True)
    rcp = 1.0 / s
    lb_out[0, :, :] = jnp.broadcast_to(t * rcp, (BBLK, K))
    ct_out[0, :, :] = jnp.broadcast_to(qg - pg * rcp, (BBLK, K))


def _topk_call(interpret=False):
    outk = lambda: jax.ShapeDtypeStruct((H, B, K), jnp.float32)
    return pl.pallas_call(
        _topk_body,
        grid=(H, B // BBLK),
        in_specs=[
            pl.BlockSpec((1, N, D), lambda h, b: (h, 0, 0)),
            pl.BlockSpec((1, BBLK, D), lambda h, b: (h, b, 0)),
            pl.BlockSpec((1, BBLK, D), lambda h, b: (h, b, 0)),
            pl.BlockSpec((BBLK, N), lambda h, b: (b, 0)),
        ],
        out_specs=[
            pl.BlockSpec((1, BBLK, K), lambda h, b: (h, b, 0)),
            pl.BlockSpec((1, BBLK, K), lambda h, b: (h, b, 0)),
        ],
        out_shape=[outk(), outk()],
        interpret=interpret,
    )


def _combine_body(ps0_ref, ps1_ref, o_ref):
    ps = 0.5 * (ps0_ref[...] + ps1_ref[...])
    o_ref[...] = jax.nn.sigmoid(jnp.mean(ps, axis=0))


def _combine_call(interpret=False):
    return pl.pallas_call(
        _combine_body,
        out_shape=jax.ShapeDtypeStruct((B,), jnp.float32),
        interpret=interpret,
    )


def kernel(pos, grads, edges, adj, label_w):
    src, dst = edges[0].astype(jnp.int32), edges[1].astype(jnp.int32)
    posT = pos.transpose(1, 0, 2)
    hh = jnp.arange(H, dtype=jnp.int32)[None, :, None]
    qidx = (jnp.stack([src, dst])[:, None, :] * H + hh).reshape(-1)
    gidx = (jnp.stack([dst, src])[:, None, :] * H + hh).reshape(-1)
    Q = _sc_gather_rows(pos.reshape(N * H, D), qidx, D, 128)
    G = _sc_gather_rows(grads.reshape(N * H, D), gidx, D, 128)
    Q = Q.reshape(2, H, B, D)
    G = G.reshape(2, H, B, D)

    adjT16 = adj.T.reshape(16 * N, N // 16)
    adj16 = adj.reshape(16 * N, N // 16)
    a0 = _sc_gather_rows(adjT16, _chunked_idx(dst, 16), N // 16, 128)
    a1 = _sc_gather_rows(adj16, _chunked_idx(src, 16), N // 16, 128)

    call = _topk_call()
    labp1, ctp1 = call(posT, Q[1], G[1], a1.reshape(B, N))
    labp0, ctp0 = call(posT, Q[0], G[0], a0.reshape(B, N))
    lw = label_w[0, 0]
    ps0 = labp0[..., 0] * lw + ctp0[..., 0]
    ps1 = labp1[..., 0] * lw + ctp1[..., 0]
    return _combine_call()(ps0, ps1)
